# baseline (speedup 1.0000x reference)
import jax
import jax.numpy as jnp
from jax.experimental import pallas as pl
from jax.experimental.pallas import tpu as pltpu

_BLK = 1000


def _lstm_kernel(x_ref, h_ref, c_ref, p_ref, out_ref, hn_ref, cn_ref):
    x = x_ref[...]
    h = h_ref[...]
    c = c_ref[...]
    wx = p_ref[0:128, :]
    wh = p_ref[128:160, :]
    bias = p_ref[160:161, :]
    wci = p_ref[168:169, 0:32]
    wcf = p_ref[176:177, 0:32]
    wco = p_ref[184:185, 0:32]
    fcw = p_ref[192:193, 0:32]
    fcb = p_ref[200:201, 0:1]
    pre = jnp.dot(x, wx, preferred_element_type=jnp.float32)
    pre = pre + jnp.dot(h, wh, preferred_element_type=jnp.float32)
    pre = pre + bias
    i_g = jax.nn.sigmoid(pre[:, 0:32] + wci * c)
    f_g = jax.nn.sigmoid(pre[:, 32:64] + wcf * c)
    t_g = jnp.tanh(pre[:, 64:96])
    c_new = f_g * c + i_g * t_g
    o_g = jax.nn.sigmoid(pre[:, 96:128] + wco * c_new)
    h_new = o_g * jnp.tanh(c_new)
    cn_ref[...] = c_new
    hn_ref[...] = h_new
    relu_h = jnp.maximum(h_new, 0.0)
    out_ref[...] = (jnp.sum(relu_h * fcw, axis=1, keepdims=True) + fcb)


def kernel(x, edge_index, edge_weight, h, c,
           W_xi, b_xi, W_hi, b_hi, W_xf, b_xf, W_hf, b_hf,
           W_xc, b_xc, W_hc, b_hc, W_xo, b_xo, W_ho, b_ho,
           w_ci, w_cf, w_co, b_i, b_f, b_c, b_o, fc_w, fc_b):
    del edge_index, edge_weight
    f_in = x.shape[1]
    h_dim = h.shape[1]
    n = x.shape[0]
    w4 = 4 * h_dim

    def row128(v):
        v = v.reshape(1, -1)
        v = jnp.pad(v, ((0, 7), (0, w4 - v.shape[1])))
        return v

    bias = jnp.concatenate([b_xi + b_hi + b_i[0],
                            b_xf + b_hf + b_f[0],
                            b_xc + b_hc + b_c[0],
                            b_xo + b_ho + b_o[0]])[None, :]
    params = jnp.concatenate([
        jnp.concatenate([W_xi, W_xf, W_xc, W_xo], axis=1),
        jnp.concatenate([W_hi, W_hf, W_hc, W_ho], axis=1),
        row128(bias),
        row128(w_ci),
        row128(w_cf),
        row128(w_co),
        row128(fc_w.T),
        row128(fc_b.reshape(1, 1)),
    ], axis=0)

    grid = (n // _BLK,)
    row = lambda i: (i, 0)
    full = lambda i: (0, 0)
    out, h_new, c_new = pl.pallas_call(
        _lstm_kernel,
        grid=grid,
        in_specs=[
            pl.BlockSpec((_BLK, f_in), row),
            pl.BlockSpec((_BLK, h_dim), row),
            pl.BlockSpec((_BLK, h_dim), row),
            pl.BlockSpec((208, w4), full),
        ],
        out_specs=[
            pl.BlockSpec((_BLK, 1), row),
            pl.BlockSpec((_BLK, h_dim), row),
            pl.BlockSpec((_BLK, h_dim), row),
        ],
        out_shape=[
            jax.ShapeDtypeStruct((n, 1), jnp.float32),
            jax.ShapeDtypeStruct((n, h_dim), jnp.float32),
            jax.ShapeDtypeStruct((n, h_dim), jnp.float32),
        ],
        compiler_params=pltpu.CompilerParams(
            dimension_semantics=("arbitrary",),
        ),
    )(x, h, c, params)
    return (out, h_new, c_new)
